# baseline (speedup 1.0000x reference)
import functools

import jax
import jax.numpy as jnp
import numpy as np
from jax import lax
from jax.experimental import pallas as pl
from jax.experimental.pallas import tpu as pltpu
from jax.experimental.pallas import tpu_sc as plsc

_N = 10000
_E = 320000
_EC = 495000
_D = 128
_H = 128
_TE = 256
_G = 100
_TL = 100.0

_NC = 2
_NS = 16
_NW = _NC * _NS

_K = 80
_RPT = 128
_EPAD = _NW * _RPT * _K
_PH = 4
_RPP = _RPT // _PH
_NPT = _N // _NS
_NA = _N + 8

_KL = 128
_CPT = 121
_TPE = _CPT * _KL
_ECP = _NW * _TPE


def _sigmoid(v):
    return 1.0 / (1.0 + jnp.exp(-v))


def _pre_body(t_ref, x_ref, tw_ref, tb_ref, w1x_ref, w1t_ref, qa_ref, qb_ref):
    targ = t_ref[...] * (1.0 / _TL)
    s = jnp.sin(targ * (np.pi / 2))
    c = jnp.cos(targ * (np.pi / 2))
    te = (s * tw_ref[0:1, :] + c * tw_ref[1:2, :] + targ * tw_ref[2:3, :]
          + tb_ref[...])
    te = te * _sigmoid(te)
    q = (jnp.dot(x_ref[...], w1x_ref[...], preferred_element_type=jnp.float32)
         + jnp.dot(te, w1t_ref[...], preferred_element_type=jnp.float32))
    qa_ref[...] = q[:, :_H]
    qb_ref[...] = q[:, _H:]


def _graph_ln(z, batch2, lnw, lnb):
    lanes = lax.broadcasted_iota(jnp.int32, (_N, 128), 1)
    oh = (batch2 == lanes).astype(jnp.float32)
    s1 = jnp.sum(z, axis=1, keepdims=True)
    s2 = jnp.sum(z * z, axis=1, keepdims=True)
    zero = jnp.zeros((), jnp.float32)
    y = (jnp.where(lanes == 0, s1, zero)
         + jnp.where(lanes == 1, s2, zero)
         + jnp.where(lanes == 2, 1.0, zero))
    stats = lax.dot_general(oh, y, (((0,), (0,)), ((), ())),
                            preferred_element_type=jnp.float32)
    den = jnp.maximum(stats[:, 2:3], 1.0) * float(_H)
    mean = stats[:, 0:1] / den
    var = stats[:, 1:2] / den - mean * mean
    inv = lax.rsqrt(var + 1e-5)
    gl = lax.broadcasted_iota(jnp.int32, (128, 128), 1)
    m2 = jnp.where(gl == 0, mean, zero) + jnp.where(gl == 1, inv, zero)
    back = jnp.dot(oh, m2, preferred_element_type=jnp.float32)
    return (z - back[:, 0:1]) * back[:, 1:2] * lnw + lnb


def _layer0_body(qa_ref, qb_ref, aa0_ref, aa1_ref, ab0_ref, ab1_ref,
                 b1_ref, w2_ref, b2_ref, lnw_ref, lnb_ref, eps_ref,
                 batch_ref, out_ref):
    g = 1.0 + eps_ref[...]
    ua = g * qa_ref[...] + aa0_ref[...] + aa1_ref[...] + b1_ref[:, :_H]
    ub = g * qb_ref[...] + ab0_ref[...] + ab1_ref[...] + b1_ref[:, _H:]
    a = jnp.maximum(ua, 0.0)
    b = jnp.maximum(ub, 0.0)
    w2 = w2_ref[...]
    z = (jnp.dot(a, w2[:_H, :], preferred_element_type=jnp.float32)
         + jnp.dot(b, w2[_H:, :], preferred_element_type=jnp.float32)
         + b2_ref[...])
    out_ref[...] = _graph_ln(z, batch_ref[...], lnw_ref[...], lnb_ref[...])


def _layer_body(final, h_ref, a0_ref, a1_ref, w1_ref, b1_ref, w2_ref, b2_ref,
                lnw_ref, lnb_ref, eps_ref, batch_ref, lw_ref, *outs):
    z0 = (1.0 + eps_ref[...]) * h_ref[...] + a0_ref[...] + a1_ref[...]
    a = jnp.maximum(
        jnp.dot(z0, w1_ref[...], preferred_element_type=jnp.float32)
        + b1_ref[...], 0.0)
    z = (jnp.dot(a, w2_ref[...], preferred_element_type=jnp.float32)
         + b2_ref[...])
    out = _graph_ln(z, batch_ref[...], lnw_ref[...], lnb_ref[...])
    outs[0][...] = out
    if final:
        outs[1][...] = out * lw_ref[...]


def _vmem_specs(n):
    return [pl.BlockSpec(memory_space=pltpu.VMEM) for _ in range(n)]


@jax.jit
def _pre(t2, x, tw, tb, w1x, w1t):
    return pl.pallas_call(
        _pre_body,
        out_shape=(jax.ShapeDtypeStruct((_N, _H), jnp.float32),
                   jax.ShapeDtypeStruct((_N, _H), jnp.float32)),
        in_specs=_vmem_specs(6),
        out_specs=tuple(_vmem_specs(2)),
    )(t2, x, tw, tb, w1x, w1t)


@jax.jit
def _layer0(qa, qb, aa0, aa1, ab0, ab1, b1, w2, b2, lnw, lnb, eps, batch2):
    return pl.pallas_call(
        _layer0_body,
        out_shape=jax.ShapeDtypeStruct((_N, _H), jnp.float32),
        in_specs=_vmem_specs(13),
        out_specs=pl.BlockSpec(memory_space=pltpu.VMEM),
    )(qa, qb, aa0, aa1, ab0, ab1, b1, w2, b2, lnw, lnb, eps, batch2)


@functools.partial(jax.jit, static_argnums=(0,))
def _layer(final, h, a0, a1, w1, b1, w2, b2, lnw, lnb, eps, batch2, lw):
    n_out = 2 if final else 1
    shp = jax.ShapeDtypeStruct((_N, _H), jnp.float32)
    return pl.pallas_call(
        functools.partial(_layer_body, final),
        out_shape=tuple([shp] * n_out),
        in_specs=_vmem_specs(12),
        out_specs=tuple(_vmem_specs(n_out)),
    )(h, a0, a1, w1, b1, w2, b2, lnw, lnb, eps, batch2, lw)


@functools.cache
def _mesh():
    return plsc.VectorSubcoreMesh(core_axis_name="c", subcore_axis_name="s",
                                  num_cores=_NC, num_subcores=_NS)


def _seg_body(table_hbm, src_hbm, dst_hbm, dep_hbm, out_hbm,
              src_v, dst_v, b0, b1, b2, b3, acc,
              g0, g1, g2, g3, s0, s1, s2, s3):
    del dep_hbm
    cid = lax.axis_index("c")
    sid = lax.axis_index("s")
    wid = cid * _NS + sid
    bufs = (b0, b1, b2, b3)
    gsem = (g0, g1, g2, g3)
    ssem = (s0, s1, s2, s3)

    zv = jnp.zeros((16,), jnp.float32)

    def zrow(i, _):
        for j in range(_H // 16):
            b0[i, pl.ds(j * 16, 16)] = zv
        return 0

    lax.fori_loop(0, _K, zrow, 0)
    for r in range(_NPT // _K):
        pltpu.sync_copy(b0, acc.at[pl.ds(sid * _NPT + r * _K, _K)])
    if _NPT % _K:
        pltpu.sync_copy(b0.at[pl.ds(0, _NPT % _K)],
                        acc.at[pl.ds(sid * _NPT + (_NPT // _K) * _K,
                                     _NPT % _K)])
    plsc.subcore_barrier()

    def gather(c, slot):
        pltpu.async_copy(table_hbm.at[src_v.at[c]], bufs[slot], gsem[slot])

    def gwait(c, slot):
        pltpu.make_async_copy(table_hbm.at[src_v.at[c]], bufs[slot],
                              gsem[slot]).wait()

    def scat(c, slot):
        pltpu.async_copy(bufs[slot], acc.at[dst_v.at[c]], ssem[slot],
                         add=True)

    def swait(slot):
        pltpu.make_async_copy(bufs[slot], acc.at[dst_v.at[0]],
                              ssem[slot]).wait()

    for ph in range(_PH):
        pltpu.sync_copy(src_hbm.at[wid, pl.ds(ph * _RPP, _RPP)], src_v)
        pltpu.sync_copy(dst_hbm.at[wid, pl.ds(ph * _RPP, _RPP)], dst_v)

        gather(0, 0)
        gather(1, 1)

        def step(c, r):
            gwait(c, r)
            scat(c, r)
            nc = c + 2
            ns = (r + 2) % 4

            @pl.when(nc < _RPP)
            def _():
                @pl.when(c >= 2)
                def _():
                    swait(ns)

                gather(nc, ns)

        def quad(q, _):
            for r in range(4):
                step(4 * q + r, r)
            return 0

        lax.fori_loop(0, _RPP // 4, quad, 0)
        for slot in range(4):
            swait(slot)

    plsc.subcore_barrier()
    pltpu.sync_copy(acc.at[pl.ds(sid * _NPT, _NPT)], out_hbm.at[cid, sid])


@jax.jit
def _seg_sum(table, src2, dst2, dep):
    f = pl.kernel(
        _seg_body,
        out_type=jax.ShapeDtypeStruct((_NC, _NS, _NPT, _H), jnp.float32),
        mesh=_mesh(),
        scratch_types=(
            [pltpu.VMEM((_RPP, _K), jnp.int32)] * 2
            + [pltpu.VMEM((_K, _H), jnp.float32)] * 4
            + [pltpu.VMEM_SHARED((_NA, _H), jnp.float32)]
            + [pltpu.SemaphoreType.DMA] * 8
        ),
    )
    return f(table, src2, dst2, dep).reshape(_NC, _N, _H)


def _link_body(h_hbm, hw_hbm, c0_hbm, c1_hbm, bias_hbm, out_hbm,
               idx0, idx1, bufa0, bufb0, bufa1, bufb1, bvec, res,
               sa0, sb0, sa1, sb1):
    cid = lax.axis_index("c")
    sid = lax.axis_index("s")
    wid = cid * _NS + sid
    pltpu.sync_copy(c0_hbm.at[wid], idx0)
    pltpu.sync_copy(c1_hbm.at[wid], idx1)
    pltpu.sync_copy(bias_hbm, bvec)
    bias = bvec[...]
    lane = lax.broadcasted_iota(jnp.int32, (16,), 0)

    def fire(j, bufa, bufb, sa, sb):
        pltpu.async_copy(h_hbm.at[idx0.at[j]], bufa, sa)
        pltpu.async_copy(hw_hbm.at[idx1.at[j]], bufb, sb)

    def drain(j, bufa, bufb, sa, sb):
        pltpu.make_async_copy(h_hbm.at[idx0.at[j]], bufa, sa).wait()
        pltpu.make_async_copy(hw_hbm.at[idx1.at[j]], bufb, sb).wait()

    def compute(j, bufa, bufb):
        def group(g, _):
            def edge(e, accv):
                row = g * 16 + e
                acc = jnp.zeros((16,), jnp.float32)
                for k in range(_H // 16):
                    acc = acc + (bufa[row, pl.ds(k * 16, 16)]
                                 * bufb[row, pl.ds(k * 16, 16)])
                return jnp.where(lane == e, jnp.sum(acc), accv)

            accv = lax.fori_loop(0, 16, edge, jnp.zeros((16,), jnp.float32))
            accv = 1.0 / (1.0 + jnp.exp(-(accv + bias)))
            res[pl.ds(j * _KL + g * 16, 16)] = accv
            return 0

        lax.fori_loop(0, _KL // 16, group, 0)

    fire(0, bufa0, bufb0, sa0, sb0)

    def pair(p, _):
        j0 = 2 * p
        j1 = j0 + 1
        drain(j0, bufa0, bufb0, sa0, sb0)

        @pl.when(j1 < _CPT)
        def _():
            fire(j1, bufa1, bufb1, sa1, sb1)

        compute(j0, bufa0, bufb0)

        @pl.when(j1 < _CPT)
        def _():
            drain(j1, bufa1, bufb1, sa1, sb1)

            @pl.when(j1 + 1 < _CPT)
            def _():
                fire(j1 + 1, bufa0, bufb0, sa0, sb0)

            compute(j1, bufa1, bufb1)

        return 0

    lax.fori_loop(0, (_CPT + 1) // 2, pair, 0)
    pltpu.sync_copy(res, out_hbm.at[wid])


@jax.jit
def _link(h, hw, c0, c1, bias16):
    f = pl.kernel(
        _link_body,
        out_type=jax.ShapeDtypeStruct((_NW, _TPE), jnp.float32),
        mesh=_mesh(),
        compiler_params=pltpu.CompilerParams(needs_layout_passes=False),
        scratch_types=[
            pltpu.VMEM((_CPT, _KL), jnp.int32),
            pltpu.VMEM((_CPT, _KL), jnp.int32),
            pltpu.VMEM((_KL, _H), jnp.float32),
            pltpu.VMEM((_KL, _H), jnp.float32),
            pltpu.VMEM((_KL, _H), jnp.float32),
            pltpu.VMEM((_KL, _H), jnp.float32),
            pltpu.VMEM((16,), jnp.float32),
            pltpu.VMEM((_TPE,), jnp.float32),
            pltpu.SemaphoreType.DMA,
            pltpu.SemaphoreType.DMA,
            pltpu.SemaphoreType.DMA,
            pltpu.SemaphoreType.DMA,
        ],
    )
    return f(h, hw, c0, c1, bias16)


def kernel(x, edge_index, batch, t, edge_cand, params):
    epad = _EPAD - _E
    src2 = jnp.pad(edge_index[0], (0, epad)).reshape(_NW, _RPT, _K)
    dst2 = jnp.pad(edge_index[1], (0, epad),
                   constant_values=_N).reshape(_NW, _RPT, _K)
    npad = _ECP - _EC
    c0 = jnp.pad(edge_cand[0], (0, npad)).reshape(_NW, _CPT, _KL)
    c1 = jnp.pad(edge_cand[1], (0, npad)).reshape(_NW, _CPT, _KL)
    t2 = t.reshape(_N, 1)
    batch2 = batch.reshape(_N, 1)

    lyr0 = params["layers"][0]
    w1x = lyr0["W1"][:_D, :]
    w1t = lyr0["W1"][_D:, :]
    qa, qb = _pre(t2, x, params["time_W"], params["time_b"].reshape(1, _TE),
                  w1x, w1t)

    dep0 = jnp.zeros((8, _H), jnp.float32)
    aqa = _seg_sum(qa, src2, dst2, dep0)
    aqb = _seg_sum(qb, src2, dst2, aqa[0, :8, :])
    h = _layer0(qa, qb, aqa[0], aqa[1], aqb[0], aqb[1],
                lyr0["b1"].reshape(1, 2 * _H), lyr0["W2"],
                lyr0["b2"].reshape(1, _H), lyr0["ln_w"].reshape(1, _H),
                lyr0["ln_b"].reshape(1, _H), lyr0["eps"].reshape(1, 1),
                batch2)

    lw = params["link_W"].reshape(1, _H)
    for i in range(1, 4):
        lyr = params["layers"][i]
        ag = _seg_sum(h, src2, dst2, dep0)
        final = (i == 3)
        outs = _layer(final, h, ag[0], ag[1], lyr["W1"],
                      lyr["b1"].reshape(1, 2 * _H), lyr["W2"],
                      lyr["b2"].reshape(1, _H), lyr["ln_w"].reshape(1, _H),
                      lyr["ln_b"].reshape(1, _H), lyr["eps"].reshape(1, 1),
                      batch2, lw)
        if final:
            h, hw = outs
        else:
            h = outs[0]

    bias16 = jnp.broadcast_to(params["link_b"].reshape(1)[0], (16,))
    probs = _link(h, hw, c0, c1, bias16)
    return probs.reshape(_ECP)[:_EC]


# speedup vs baseline: 2.3170x; 2.3170x over previous
import functools

import jax
import jax.numpy as jnp
import numpy as np
from jax import lax
from jax.experimental import pallas as pl
from jax.experimental.pallas import tpu as pltpu
from jax.experimental.pallas import tpu_sc as plsc

_N = 10000
_E = 320000
_EC = 495000
_D = 128
_H = 128
_TE = 256
_G = 100
_TL = 100.0

_NC = 2
_NS = 16
_NW = _NC * _NS

_K = 125
_RPT = _E // (_NW * _K)
_PH = 2
_RPP = _RPT // _PH
_NPT = _N // _NS

_KL = 128
_CPT = 121
_TPE = _CPT * _KL
_ECP = _NW * _TPE


def _sigmoid(v):
    return 1.0 / (1.0 + jnp.exp(-v))


def _pre_body(t_ref, x_ref, tw_ref, tb_ref, w1x_ref, w1t_ref, qa_ref, qb_ref):
    targ = t_ref[...] * (1.0 / _TL)
    s = jnp.sin(targ * (np.pi / 2))
    c = jnp.cos(targ * (np.pi / 2))
    te = (s * tw_ref[0:1, :] + c * tw_ref[1:2, :] + targ * tw_ref[2:3, :]
          + tb_ref[...])
    te = te * _sigmoid(te)
    q = (jnp.dot(x_ref[...], w1x_ref[...], preferred_element_type=jnp.float32)
         + jnp.dot(te, w1t_ref[...], preferred_element_type=jnp.float32))
    qa_ref[...] = q[:, :_H]
    qb_ref[...] = q[:, _H:]


def _graph_ln(z, batch2, lnw, lnb):
    lanes = lax.broadcasted_iota(jnp.int32, (_N, 128), 1)
    oh = (batch2 == lanes).astype(jnp.float32)
    s1 = jnp.sum(z, axis=1, keepdims=True)
    s2 = jnp.sum(z * z, axis=1, keepdims=True)
    zero = jnp.zeros((), jnp.float32)
    y = (jnp.where(lanes == 0, s1, zero)
         + jnp.where(lanes == 1, s2, zero)
         + jnp.where(lanes == 2, 1.0, zero))
    stats = lax.dot_general(oh, y, (((0,), (0,)), ((), ())),
                            preferred_element_type=jnp.float32)
    den = jnp.maximum(stats[:, 2:3], 1.0) * float(_H)
    mean = stats[:, 0:1] / den
    var = stats[:, 1:2] / den - mean * mean
    inv = lax.rsqrt(var + 1e-5)
    gl = lax.broadcasted_iota(jnp.int32, (128, 128), 1)
    m2 = jnp.where(gl == 0, mean, zero) + jnp.where(gl == 1, inv, zero)
    back = jnp.dot(oh, m2, preferred_element_type=jnp.float32)
    return (z - back[:, 0:1]) * back[:, 1:2] * lnw + lnb


def _layer0_body(qa_ref, qb_ref, aa0_ref, aa1_ref, ab0_ref, ab1_ref,
                 b1_ref, w2_ref, b2_ref, lnw_ref, lnb_ref, eps_ref,
                 batch_ref, out_ref):
    g = 1.0 + eps_ref[...]
    ua = g * qa_ref[...] + aa0_ref[...] + aa1_ref[...] + b1_ref[:, :_H]
    ub = g * qb_ref[...] + ab0_ref[...] + ab1_ref[...] + b1_ref[:, _H:]
    a = jnp.maximum(ua, 0.0)
    b = jnp.maximum(ub, 0.0)
    w2 = w2_ref[...]
    z = (jnp.dot(a, w2[:_H, :], preferred_element_type=jnp.float32)
         + jnp.dot(b, w2[_H:, :], preferred_element_type=jnp.float32)
         + b2_ref[...])
    out_ref[...] = _graph_ln(z, batch_ref[...], lnw_ref[...], lnb_ref[...])


def _layer_body(final, h_ref, a0_ref, a1_ref, w1_ref, b1_ref, w2_ref, b2_ref,
                lnw_ref, lnb_ref, eps_ref, batch_ref, lw_ref, *outs):
    z0 = (1.0 + eps_ref[...]) * h_ref[...] + a0_ref[...] + a1_ref[...]
    a = jnp.maximum(
        jnp.dot(z0, w1_ref[...], preferred_element_type=jnp.float32)
        + b1_ref[...], 0.0)
    z = (jnp.dot(a, w2_ref[...], preferred_element_type=jnp.float32)
         + b2_ref[...])
    out = _graph_ln(z, batch_ref[...], lnw_ref[...], lnb_ref[...])
    outs[0][...] = out
    if final:
        outs[1][...] = out * lw_ref[...]


def _vmem_specs(n):
    return [pl.BlockSpec(memory_space=pltpu.VMEM) for _ in range(n)]


@jax.jit
def _pre(t2, x, tw, tb, w1x, w1t):
    return pl.pallas_call(
        _pre_body,
        out_shape=(jax.ShapeDtypeStruct((_N, _H), jnp.float32),
                   jax.ShapeDtypeStruct((_N, _H), jnp.float32)),
        in_specs=_vmem_specs(6),
        out_specs=tuple(_vmem_specs(2)),
    )(t2, x, tw, tb, w1x, w1t)


@jax.jit
def _layer0(qa, qb, aa0, aa1, ab0, ab1, b1, w2, b2, lnw, lnb, eps, batch2):
    return pl.pallas_call(
        _layer0_body,
        out_shape=jax.ShapeDtypeStruct((_N, _H), jnp.float32),
        in_specs=_vmem_specs(13),
        out_specs=pl.BlockSpec(memory_space=pltpu.VMEM),
    )(qa, qb, aa0, aa1, ab0, ab1, b1, w2, b2, lnw, lnb, eps, batch2)


@functools.partial(jax.jit, static_argnums=(0,))
def _layer(final, h, a0, a1, w1, b1, w2, b2, lnw, lnb, eps, batch2, lw):
    n_out = 2 if final else 1
    shp = jax.ShapeDtypeStruct((_N, _H), jnp.float32)
    return pl.pallas_call(
        functools.partial(_layer_body, final),
        out_shape=tuple([shp] * n_out),
        in_specs=_vmem_specs(12),
        out_specs=tuple(_vmem_specs(n_out)),
    )(h, a0, a1, w1, b1, w2, b2, lnw, lnb, eps, batch2, lw)


@functools.cache
def _mesh():
    return plsc.VectorSubcoreMesh(core_axis_name="c", subcore_axis_name="s",
                                  num_cores=_NC, num_subcores=_NS)


def _seg_body(table_hbm, src_hbm, dst_hbm, dep_hbm, out_hbm,
              src_v, dst_v, buf, buf2, acc, sem, sem2):
    del dep_hbm
    cid = lax.axis_index("c")
    sid = lax.axis_index("s")
    wid = cid * _NS + sid

    zv = jnp.zeros((16,), jnp.float32)

    def zrow(i, _):
        for j in range(_H // 16):
            buf[i, pl.ds(j * 16, 16)] = zv
        return 0

    lax.fori_loop(0, _K, zrow, 0)
    for r in range(_NPT // _K):
        pltpu.sync_copy(buf, acc.at[pl.ds(sid * _NPT + r * _K, _K)])
    plsc.subcore_barrier()

    for ph in range(_PH):
        pltpu.sync_copy(src_hbm.at[wid, pl.ds(ph * _RPP, _RPP)], src_v)
        pltpu.sync_copy(dst_hbm.at[wid, pl.ds(ph * _RPP, _RPP)], dst_v)

        pltpu.async_copy(table_hbm.at[src_v.at[0]], buf, sem)

        def pair(p, _):
            j0 = 2 * p
            j1 = j0 + 1
            pltpu.make_async_copy(table_hbm.at[src_v.at[j0]], buf, sem).wait()
            pltpu.async_copy(table_hbm.at[src_v.at[j1]], buf2, sem2)
            pltpu.sync_copy(buf, acc.at[dst_v.at[j0]], add=True)
            pltpu.make_async_copy(table_hbm.at[src_v.at[j1]], buf2,
                                  sem2).wait()

            @pl.when(p < _RPP // 2 - 1)
            def _():
                pltpu.async_copy(table_hbm.at[src_v.at[j1 + 1]], buf, sem)

            pltpu.sync_copy(buf2, acc.at[dst_v.at[j1]], add=True)
            return 0

        lax.fori_loop(0, _RPP // 2, pair, 0)

    plsc.subcore_barrier()
    pltpu.sync_copy(acc.at[pl.ds(sid * _NPT, _NPT)], out_hbm.at[cid, sid])


@jax.jit
def _seg_sum(table, src2, dst2, dep):
    f = pl.kernel(
        _seg_body,
        out_type=jax.ShapeDtypeStruct((_NC, _NS, _NPT, _H), jnp.float32),
        mesh=_mesh(),
        scratch_types=[
            pltpu.VMEM((_RPP, _K), jnp.int32),
            pltpu.VMEM((_RPP, _K), jnp.int32),
            pltpu.VMEM((_K, _H), jnp.float32),
            pltpu.VMEM((_K, _H), jnp.float32),
            pltpu.VMEM_SHARED((_N, _H), jnp.float32),
            pltpu.SemaphoreType.DMA,
            pltpu.SemaphoreType.DMA,
        ],
    )
    return f(table, src2, dst2, dep).reshape(_NC, _N, _H)


def _link_body(h_hbm, hw_hbm, c0_hbm, c1_hbm, bias_hbm, out_hbm,
               idx0, idx1, bufa0, bufb0, bufa1, bufb1, bvec, res,
               sa0, sb0, sa1, sb1):
    cid = lax.axis_index("c")
    sid = lax.axis_index("s")
    wid = cid * _NS + sid
    pltpu.sync_copy(c0_hbm.at[wid], idx0)
    pltpu.sync_copy(c1_hbm.at[wid], idx1)
    pltpu.sync_copy(bias_hbm, bvec)
    bias = bvec[...]
    lane = lax.broadcasted_iota(jnp.int32, (16,), 0)

    def fire(j, bufa, bufb, sa, sb):
        pltpu.async_copy(h_hbm.at[idx0.at[j]], bufa, sa)
        pltpu.async_copy(hw_hbm.at[idx1.at[j]], bufb, sb)

    def drain(j, bufa, bufb, sa, sb):
        pltpu.make_async_copy(h_hbm.at[idx0.at[j]], bufa, sa).wait()
        pltpu.make_async_copy(hw_hbm.at[idx1.at[j]], bufb, sb).wait()

    def compute(j, bufa, bufb):
        def group(g, _):
            def edge(e, accv):
                row = g * 16 + e
                acc = jnp.zeros((16,), jnp.float32)
                for k in range(_H // 16):
                    acc = acc + (bufa[row, pl.ds(k * 16, 16)]
                                 * bufb[row, pl.ds(k * 16, 16)])
                return jnp.where(lane == e, jnp.sum(acc), accv)

            accv = lax.fori_loop(0, 16, edge, jnp.zeros((16,), jnp.float32))
            accv = 1.0 / (1.0 + jnp.exp(-(accv + bias)))
            res[pl.ds(j * _KL + g * 16, 16)] = accv
            return 0

        lax.fori_loop(0, _KL // 16, group, 0)

    fire(0, bufa0, bufb0, sa0, sb0)

    def pair(p, _):
        j0 = 2 * p
        j1 = j0 + 1
        drain(j0, bufa0, bufb0, sa0, sb0)

        @pl.when(j1 < _CPT)
        def _():
            fire(j1, bufa1, bufb1, sa1, sb1)

        compute(j0, bufa0, bufb0)

        @pl.when(j1 < _CPT)
        def _():
            drain(j1, bufa1, bufb1, sa1, sb1)

            @pl.when(j1 + 1 < _CPT)
            def _():
                fire(j1 + 1, bufa0, bufb0, sa0, sb0)

            compute(j1, bufa1, bufb1)

        return 0

    lax.fori_loop(0, (_CPT + 1) // 2, pair, 0)
    pltpu.sync_copy(res, out_hbm.at[wid])


@jax.jit
def _link(h, hw, c0, c1, bias16):
    f = pl.kernel(
        _link_body,
        out_type=jax.ShapeDtypeStruct((_NW, _TPE), jnp.float32),
        mesh=_mesh(),
        compiler_params=pltpu.CompilerParams(needs_layout_passes=False),
        scratch_types=[
            pltpu.VMEM((_CPT, _KL), jnp.int32),
            pltpu.VMEM((_CPT, _KL), jnp.int32),
            pltpu.VMEM((_KL, _H), jnp.float32),
            pltpu.VMEM((_KL, _H), jnp.float32),
            pltpu.VMEM((_KL, _H), jnp.float32),
            pltpu.VMEM((_KL, _H), jnp.float32),
            pltpu.VMEM((16,), jnp.float32),
            pltpu.VMEM((_TPE,), jnp.float32),
            pltpu.SemaphoreType.DMA,
            pltpu.SemaphoreType.DMA,
            pltpu.SemaphoreType.DMA,
            pltpu.SemaphoreType.DMA,
        ],
    )
    return f(h, hw, c0, c1, bias16)


def kernel(x, edge_index, batch, t, edge_cand, params):
    src2 = edge_index[0].reshape(_NW, _RPT, _K)
    dst2 = edge_index[1].reshape(_NW, _RPT, _K)
    npad = _ECP - _EC
    c0 = jnp.pad(edge_cand[0], (0, npad)).reshape(_NW, _CPT, _KL)
    c1 = jnp.pad(edge_cand[1], (0, npad)).reshape(_NW, _CPT, _KL)
    t2 = t.reshape(_N, 1)
    batch2 = batch.reshape(_N, 1)

    lyr0 = params["layers"][0]
    w1x = lyr0["W1"][:_D, :]
    w1t = lyr0["W1"][_D:, :]
    qa, qb = _pre(t2, x, params["time_W"], params["time_b"].reshape(1, _TE),
                  w1x, w1t)

    dep0 = jnp.zeros((8, _H), jnp.float32)
    aqa = _seg_sum(qa, src2, dst2, dep0)
    aqb = _seg_sum(qb, src2, dst2, aqa[0, :8, :])
    h = _layer0(qa, qb, aqa[0], aqa[1], aqb[0], aqb[1],
                lyr0["b1"].reshape(1, 2 * _H), lyr0["W2"],
                lyr0["b2"].reshape(1, _H), lyr0["ln_w"].reshape(1, _H),
                lyr0["ln_b"].reshape(1, _H), lyr0["eps"].reshape(1, 1),
                batch2)

    lw = params["link_W"].reshape(1, _H)
    for i in range(1, 4):
        lyr = params["layers"][i]
        ag = _seg_sum(h, src2, dst2, dep0)
        final = (i == 3)
        outs = _layer(final, h, ag[0], ag[1], lyr["W1"],
                      lyr["b1"].reshape(1, 2 * _H), lyr["W2"],
                      lyr["b2"].reshape(1, _H), lyr["ln_w"].reshape(1, _H),
                      lyr["ln_b"].reshape(1, _H), lyr["eps"].reshape(1, 1),
                      batch2, lw)
        if final:
            h, hw = outs
        else:
            h = outs[0]

    bias16 = jnp.broadcast_to(params["link_b"].reshape(1)[0], (16,))
    probs = _link(h, hw, c0, c1, bias16)
    return probs.reshape(_ECP)[:_EC]
